# baseline (speedup 1.0000x reference)
import functools

import jax
import jax.numpy as jnp
from jax import lax
from jax.experimental import pallas as pl
from jax.experimental.pallas import tpu as pltpu
from jax.experimental.pallas import tpu_sc as plsc

_SCALING = 32.0 / 16.0
_E = 8
_R = 16
_NW = 32
_LANES = 16


def _router_body(x_ref, rwpad_ref, rw8_ref, logits_ref, logits_t_ref):
    xt = x_ref[...]
    logits_ref[...] = jnp.dot(xt, rwpad_ref[...].T,
                              preferred_element_type=jnp.float32)
    logits_t_ref[...] = jax.lax.dot_general(
        rw8_ref[...], xt, (((1,), (1,)), ((), ())),
        preferred_element_type=jnp.float32)


def _sc_routing_body(lg_hbm, w_hbm, lv, wv):
    chunk = lv.shape[1]
    wid = lax.axis_index("s") * 2 + lax.axis_index("c")
    base = wid * chunk
    pltpu.sync_copy(lg_hbm.at[:, pl.ds(base, chunk)], lv)
    for v in range(chunk // _LANES):
        sl = pl.ds(v * _LANES, _LANES)
        ls = [lv[e, sl] for e in range(_E)]
        m1 = ls[0]
        for e in range(1, _E):
            m1 = jnp.maximum(m1, ls[e])
        big = 1 << 30
        i1 = jnp.where(ls[0] == m1, 0, big)
        for e in range(1, _E):
            i1 = jnp.minimum(i1, jnp.where(ls[e] == m1, e, big))
        neg = jnp.float32(-3.0e38)
        l2 = [jnp.where(i1 == e, neg, ls[e]) for e in range(_E)]
        m2 = l2[0]
        for e in range(1, _E):
            m2 = jnp.maximum(m2, l2[e])
        i2 = jnp.where(l2[0] == m2, 0, big)
        for e in range(1, _E):
            i2 = jnp.minimum(i2, jnp.where(l2[e] == m2, e, big))
        s1 = 1.0 / (1.0 + jnp.exp(m2 - m1))
        s2 = 1.0 - s1
        for e in range(_E):
            w = jnp.where(i1 == e, s1, 0.0) + jnp.where(i2 == e, s2, 0.0)
            wv[e, sl] = w * _SCALING
    pltpu.sync_copy(wv, w_hbm.at[:, pl.ds(base, chunk)])


def _mlp_body(x_ref, acat_ref, w1_ref, w3_ref, w2_ref, shared_ref, a_ref):
    f = pl.program_id(1)
    xt = x_ref[...]

    @pl.when(f == 0)
    def _lora_a():
        a_ref[...] = jnp.dot(xt, acat_ref[...].T,
                             preferred_element_type=jnp.float32)
        shared_ref[...] = jnp.zeros_like(shared_ref)

    dn = (((1,), (1,)), ((), ()))
    h1 = jax.lax.dot_general(xt, w1_ref[...], dn,
                             preferred_element_type=jnp.float32)
    h3 = jax.lax.dot_general(xt, w3_ref[...], dn,
                             preferred_element_type=jnp.float32)
    h = jax.nn.silu(h1) * h3
    shared_ref[...] += jax.lax.dot_general(h, w2_ref[...], dn,
                                           preferred_element_type=jnp.float32)


def _combine_body(shared_ref, a_ref, w8_ref, p_ref, bcat_ref, out_ref):
    wl = jax.lax.dot_general(w8_ref[...], p_ref[...],
                             (((0,), (0,)), ((), ())),
                             preferred_element_type=jnp.float32)
    lora = jax.lax.dot_general(a_ref[...] * wl, bcat_ref[...],
                               (((1,), (0,)), ((), ())),
                               preferred_element_type=jnp.float32)
    out_ref[...] = shared_ref[...] + lora


@jax.jit
def _moe_block(x, rw_pad, rw8, a_cat, b_cat, w1, w2, w3):
    t, h = x.shape
    ffn = w1.shape[0]
    tm, fk = 1024, 1024
    nt, nf = t // tm, ffn // fk

    logits_pad, logits_t = pl.pallas_call(
        _router_body,
        grid=(nt,),
        in_specs=[
            pl.BlockSpec((tm, h), lambda t_: (t_, 0)),
            pl.BlockSpec((128, h), lambda t_: (0, 0)),
            pl.BlockSpec((_E, h), lambda t_: (0, 0)),
        ],
        out_specs=[
            pl.BlockSpec((tm, 128), lambda t_: (t_, 0)),
            pl.BlockSpec((_E, tm), lambda t_: (0, t_)),
        ],
        out_shape=[
            jax.ShapeDtypeStruct((t, 128), jnp.float32),
            jax.ShapeDtypeStruct((_E, t), jnp.float32),
        ],
    )(x, rw_pad, rw8)

    chunk = t // _NW
    mesh = plsc.VectorSubcoreMesh(core_axis_name="c", subcore_axis_name="s")
    w8 = pl.kernel(
        _sc_routing_body,
        out_type=jax.ShapeDtypeStruct((_E, t), jnp.float32),
        mesh=mesh,
        scratch_types=[
            pltpu.VMEM((_E, chunk), jnp.float32),
            pltpu.VMEM((_E, chunk), jnp.float32),
        ],
    )(logits_t)

    shared, a = pl.pallas_call(
        _mlp_body,
        grid=(nt, nf),
        in_specs=[
            pl.BlockSpec((tm, h), lambda t_, f_: (t_, 0)),
            pl.BlockSpec((128, h), lambda t_, f_: (0, 0)),
            pl.BlockSpec((fk, h), lambda t_, f_: (f_, 0)),
            pl.BlockSpec((fk, h), lambda t_, f_: (f_, 0)),
            pl.BlockSpec((h, fk), lambda t_, f_: (0, f_)),
        ],
        out_specs=[
            pl.BlockSpec((tm, h), lambda t_, f_: (t_, 0)),
            pl.BlockSpec((tm, 128), lambda t_, f_: (t_, 0)),
        ],
        out_shape=[
            jax.ShapeDtypeStruct((t, h), jnp.float32),
            jax.ShapeDtypeStruct((t, 128), jnp.float32),
        ],
        compiler_params=pltpu.CompilerParams(
            dimension_semantics=("parallel", "arbitrary"),
        ),
    )(x, a_cat, w1, w3, w2)

    p_sel = jnp.repeat(jnp.eye(_E, dtype=jnp.float32), _R, axis=1)
    out = pl.pallas_call(
        _combine_body,
        grid=(nt,),
        in_specs=[
            pl.BlockSpec((tm, h), lambda t_: (t_, 0)),
            pl.BlockSpec((tm, 128), lambda t_: (t_, 0)),
            pl.BlockSpec((_E, tm), lambda t_: (0, t_)),
            pl.BlockSpec((_E, 128), lambda t_: (0, 0)),
            pl.BlockSpec((128, h), lambda t_: (0, 0)),
        ],
        out_specs=pl.BlockSpec((tm, h), lambda t_: (t_, 0)),
        out_shape=jax.ShapeDtypeStruct((t, h), jnp.float32),
    )(shared, a, w8, p_sel, b_cat)
    return out, logits_pad


def kernel(hidden_states, router_w, w1, w2, w3, lora_A, lora_B):
    b, s, h = hidden_states.shape
    x = hidden_states.reshape(-1, h)
    e, r = lora_A.shape[0], lora_A.shape[1]
    a_cat = lora_A.reshape(e * r, h)
    b_cat = lora_B.transpose(0, 2, 1).reshape(e * r, h)
    rw_pad = jnp.zeros((128, h), x.dtype).at[:e].set(router_w)
    out, logits_pad = _moe_block(x, rw_pad, router_w, a_cat, b_cat, w1, w2, w3)
    return out.reshape(b, s, h), logits_pad[:, :e]


# speedup vs baseline: 1.0130x; 1.0130x over previous
import functools

import jax
import jax.numpy as jnp
from jax import lax
from jax.experimental import pallas as pl
from jax.experimental.pallas import tpu as pltpu
from jax.experimental.pallas import tpu_sc as plsc

_SCALING = 32.0 / 16.0
_E = 8
_R = 16
_NW = 32
_LANES = 16


def _router_body(x_ref, rwpad_ref, rw8_ref, logits_ref, logits_t_ref):
    xt = x_ref[...]
    logits_ref[...] = jnp.dot(xt, rwpad_ref[...].T,
                              preferred_element_type=jnp.float32)
    logits_t_ref[...] = jax.lax.dot_general(
        rw8_ref[...], xt, (((1,), (1,)), ((), ())),
        preferred_element_type=jnp.float32)


def _sc_routing_body(lg_hbm, w_hbm, lv, wv):
    chunk = lv.shape[1]
    wid = lax.axis_index("s") * 2 + lax.axis_index("c")
    base = wid * chunk
    pltpu.sync_copy(lg_hbm.at[:, pl.ds(base, chunk)], lv)
    for v in range(chunk // _LANES):
        sl = pl.ds(v * _LANES, _LANES)
        ls = [lv[e, sl] for e in range(_E)]
        m1 = ls[0]
        for e in range(1, _E):
            m1 = jnp.maximum(m1, ls[e])
        big = 1 << 30
        i1 = jnp.where(ls[0] == m1, 0, big)
        for e in range(1, _E):
            i1 = jnp.minimum(i1, jnp.where(ls[e] == m1, e, big))
        neg = jnp.float32(-3.0e38)
        l2 = [jnp.where(i1 == e, neg, ls[e]) for e in range(_E)]
        m2 = l2[0]
        for e in range(1, _E):
            m2 = jnp.maximum(m2, l2[e])
        i2 = jnp.where(l2[0] == m2, 0, big)
        for e in range(1, _E):
            i2 = jnp.minimum(i2, jnp.where(l2[e] == m2, e, big))
        s1 = 1.0 / (1.0 + jnp.exp(m2 - m1))
        s2 = 1.0 - s1
        for e in range(_E):
            w = jnp.where(i1 == e, s1, 0.0) + jnp.where(i2 == e, s2, 0.0)
            wv[e, sl] = w * _SCALING
    pltpu.sync_copy(wv, w_hbm.at[:, pl.ds(base, chunk)])


def _mlp_body(x_ref, acat_ref, w1_ref, w3_ref, w2_ref, w8_ref, p_ref,
              bcat_ref, out_ref, a_scr):
    f = pl.program_id(1)
    nf = pl.num_programs(1)
    xt = x_ref[...]

    @pl.when(f == 0)
    def _lora_a():
        a_scr[...] = jnp.dot(xt, acat_ref[...].T,
                             preferred_element_type=jnp.float32)

    dn = (((1,), (1,)), ((), ()))
    h1 = jax.lax.dot_general(xt, w1_ref[...], dn,
                             preferred_element_type=jnp.float32)
    h3 = jax.lax.dot_general(xt, w3_ref[...], dn,
                             preferred_element_type=jnp.float32)
    h = jax.nn.silu(h1) * h3
    contrib = jax.lax.dot_general(h, w2_ref[...], dn,
                                  preferred_element_type=jnp.float32)

    @pl.when(f == 0)
    def _init():
        out_ref[...] = contrib

    @pl.when(f != 0)
    def _acc():
        out_ref[...] += contrib

    @pl.when(f == nf - 1)
    def _combine():
        wl = jax.lax.dot_general(w8_ref[...], p_ref[...],
                                 (((0,), (0,)), ((), ())),
                                 preferred_element_type=jnp.float32)
        out_ref[...] += jax.lax.dot_general(
            a_scr[...] * wl, bcat_ref[...], (((1,), (0,)), ((), ())),
            preferred_element_type=jnp.float32)


@jax.jit
def _moe_block(x, rw_pad, rw8, a_cat, b_cat, w1, w2, w3):
    t, h = x.shape
    ffn = w1.shape[0]
    tm, fk = 1024, 1024
    nt, nf = t // tm, ffn // fk

    logits_pad, logits_t = pl.pallas_call(
        _router_body,
        grid=(nt,),
        in_specs=[
            pl.BlockSpec((tm, h), lambda t_: (t_, 0)),
            pl.BlockSpec((128, h), lambda t_: (0, 0)),
            pl.BlockSpec((_E, h), lambda t_: (0, 0)),
        ],
        out_specs=[
            pl.BlockSpec((tm, 128), lambda t_: (t_, 0)),
            pl.BlockSpec((_E, tm), lambda t_: (0, t_)),
        ],
        out_shape=[
            jax.ShapeDtypeStruct((t, 128), jnp.float32),
            jax.ShapeDtypeStruct((_E, t), jnp.float32),
        ],
    )(x, rw_pad, rw8)

    chunk = t // _NW
    mesh = plsc.VectorSubcoreMesh(core_axis_name="c", subcore_axis_name="s")
    w8 = pl.kernel(
        _sc_routing_body,
        out_type=jax.ShapeDtypeStruct((_E, t), jnp.float32),
        mesh=mesh,
        scratch_types=[
            pltpu.VMEM((_E, chunk), jnp.float32),
            pltpu.VMEM((_E, chunk), jnp.float32),
        ],
    )(logits_t)

    p_sel = jnp.repeat(jnp.eye(_E, dtype=jnp.float32), _R, axis=1)
    out = pl.pallas_call(
        _mlp_body,
        grid=(nt, nf),
        in_specs=[
            pl.BlockSpec((tm, h), lambda t_, f_: (t_, 0)),
            pl.BlockSpec((128, h), lambda t_, f_: (0, 0)),
            pl.BlockSpec((fk, h), lambda t_, f_: (f_, 0)),
            pl.BlockSpec((fk, h), lambda t_, f_: (f_, 0)),
            pl.BlockSpec((h, fk), lambda t_, f_: (0, f_)),
            pl.BlockSpec((_E, tm), lambda t_, f_: (0, t_)),
            pl.BlockSpec((_E, 128), lambda t_, f_: (0, 0)),
            pl.BlockSpec((128, h), lambda t_, f_: (0, 0)),
        ],
        out_specs=pl.BlockSpec((tm, h), lambda t_, f_: (t_, 0)),
        out_shape=jax.ShapeDtypeStruct((t, h), jnp.float32),
        scratch_shapes=[pltpu.VMEM((tm, 128), jnp.float32)],
        compiler_params=pltpu.CompilerParams(
            dimension_semantics=("parallel", "arbitrary"),
        ),
    )(x, a_cat, w1, w3, w2, w8, p_sel, b_cat)
    return out, logits_pad


def kernel(hidden_states, router_w, w1, w2, w3, lora_A, lora_B):
    b, s, h = hidden_states.shape
    x = hidden_states.reshape(-1, h)
    e, r = lora_A.shape[0], lora_A.shape[1]
    a_cat = lora_A.reshape(e * r, h)
    b_cat = lora_B.transpose(0, 2, 1).reshape(e * r, h)
    rw_pad = jnp.zeros((128, h), x.dtype).at[:e].set(router_w)
    out, logits_pad = _moe_block(x, rw_pad, router_w, a_cat, b_cat, w1, w2, w3)
    return out.reshape(b, s, h), logits_pad[:, :e]


# speedup vs baseline: 1.0258x; 1.0126x over previous
import functools

import jax
import jax.numpy as jnp
from jax import lax
from jax.experimental import pallas as pl
from jax.experimental.pallas import tpu as pltpu
from jax.experimental.pallas import tpu_sc as plsc

_SCALING = 32.0 / 16.0
_E = 8
_R = 16
_NW = 32
_LANES = 16


def _router_body(x_ref, rw8_ref, logits_ref, logits_t_ref):
    xt = x_ref[...]
    logits_t_ref[...] = jax.lax.dot_general(
        rw8_ref[...], xt, (((1,), (1,)), ((), ())),
        preferred_element_type=jnp.float32)
    logits_ref[...] = jax.lax.dot_general(
        xt, rw8_ref[...], (((1,), (1,)), ((), ())),
        preferred_element_type=jnp.float32)


def _sc_routing_body(lg_hbm, w_hbm, lv, wv):
    chunk = lv.shape[1]
    wid = lax.axis_index("s") * 2 + lax.axis_index("c")
    base = wid * chunk
    pltpu.sync_copy(lg_hbm.at[:, pl.ds(base, chunk)], lv)
    for v in range(chunk // _LANES):
        sl = pl.ds(v * _LANES, _LANES)
        ls = [lv[e, sl] for e in range(_E)]
        m1 = ls[0]
        for e in range(1, _E):
            m1 = jnp.maximum(m1, ls[e])
        big = 1 << 30
        i1 = jnp.where(ls[0] == m1, 0, big)
        for e in range(1, _E):
            i1 = jnp.minimum(i1, jnp.where(ls[e] == m1, e, big))
        neg = jnp.float32(-3.0e38)
        l2 = [jnp.where(i1 == e, neg, ls[e]) for e in range(_E)]
        m2 = l2[0]
        for e in range(1, _E):
            m2 = jnp.maximum(m2, l2[e])
        i2 = jnp.where(l2[0] == m2, 0, big)
        for e in range(1, _E):
            i2 = jnp.minimum(i2, jnp.where(l2[e] == m2, e, big))
        s1 = 1.0 / (1.0 + jnp.exp(m2 - m1))
        s2 = 1.0 - s1
        for e in range(_E):
            w = jnp.where(i1 == e, s1, 0.0) + jnp.where(i2 == e, s2, 0.0)
            wv[e, sl] = w * _SCALING
    pltpu.sync_copy(wv, w_hbm.at[:, pl.ds(base, chunk)])


def _mlp_body(x_ref, acat_ref, w1_ref, w3_ref, w2_ref, w8_ref, p_ref,
              bcat_ref, out_ref, a_scr):
    f = pl.program_id(1)
    nf = pl.num_programs(1)
    xt = x_ref[...]

    @pl.when(f == 0)
    def _lora_a():
        a_scr[...] = jnp.dot(xt, acat_ref[...].T,
                             preferred_element_type=jnp.float32)

    dn = (((1,), (1,)), ((), ()))
    h1 = jax.lax.dot_general(xt, w1_ref[...], dn,
                             preferred_element_type=jnp.float32)
    h3 = jax.lax.dot_general(xt, w3_ref[...], dn,
                             preferred_element_type=jnp.float32)
    h = jax.nn.silu(h1) * h3
    contrib = jax.lax.dot_general(h, w2_ref[...], dn,
                                  preferred_element_type=jnp.float32)

    @pl.when(f == 0)
    def _init():
        out_ref[...] = contrib

    @pl.when(f != 0)
    def _acc():
        out_ref[...] += contrib

    @pl.when(f == nf - 1)
    def _combine():
        wl = jax.lax.dot_general(w8_ref[...], p_ref[...],
                                 (((0,), (0,)), ((), ())),
                                 preferred_element_type=jnp.float32)
        out_ref[...] += jax.lax.dot_general(
            a_scr[...] * wl, bcat_ref[...], (((1,), (0,)), ((), ())),
            preferred_element_type=jnp.float32)


@jax.jit
def _moe_block(x, rw8, a_cat, b_cat, w1, w2, w3):
    t, h = x.shape
    ffn = w1.shape[0]
    tm, fk = 1024, 1024
    nt, nf = t // tm, ffn // fk

    logits, logits_t = pl.pallas_call(
        _router_body,
        grid=(nt,),
        in_specs=[
            pl.BlockSpec((tm, h), lambda t_: (t_, 0)),
            pl.BlockSpec((_E, h), lambda t_: (0, 0)),
        ],
        out_specs=[
            pl.BlockSpec((tm, _E), lambda t_: (t_, 0)),
            pl.BlockSpec((_E, tm), lambda t_: (0, t_)),
        ],
        out_shape=[
            jax.ShapeDtypeStruct((t, _E), jnp.float32),
            jax.ShapeDtypeStruct((_E, t), jnp.float32),
        ],
    )(x, rw8)

    chunk = t // _NW
    mesh = plsc.VectorSubcoreMesh(core_axis_name="c", subcore_axis_name="s")
    w8 = pl.kernel(
        _sc_routing_body,
        out_type=jax.ShapeDtypeStruct((_E, t), jnp.float32),
        mesh=mesh,
        scratch_types=[
            pltpu.VMEM((_E, chunk), jnp.float32),
            pltpu.VMEM((_E, chunk), jnp.float32),
        ],
    )(logits_t)

    p_sel = jnp.repeat(jnp.eye(_E, dtype=jnp.float32), _R, axis=1)
    out = pl.pallas_call(
        _mlp_body,
        grid=(nt, nf),
        in_specs=[
            pl.BlockSpec((tm, h), lambda t_, f_: (t_, 0)),
            pl.BlockSpec((128, h), lambda t_, f_: (0, 0)),
            pl.BlockSpec((fk, h), lambda t_, f_: (f_, 0)),
            pl.BlockSpec((fk, h), lambda t_, f_: (f_, 0)),
            pl.BlockSpec((h, fk), lambda t_, f_: (0, f_)),
            pl.BlockSpec((_E, tm), lambda t_, f_: (0, t_)),
            pl.BlockSpec((_E, 128), lambda t_, f_: (0, 0)),
            pl.BlockSpec((128, h), lambda t_, f_: (0, 0)),
        ],
        out_specs=pl.BlockSpec((tm, h), lambda t_, f_: (t_, 0)),
        out_shape=jax.ShapeDtypeStruct((t, h), jnp.float32),
        scratch_shapes=[pltpu.VMEM((tm, 128), jnp.float32)],
        compiler_params=pltpu.CompilerParams(
            dimension_semantics=("parallel", "arbitrary"),
        ),
    )(x, a_cat, w1, w3, w2, w8, p_sel, b_cat)
    return out, logits


def kernel(hidden_states, router_w, w1, w2, w3, lora_A, lora_B):
    b, s, h = hidden_states.shape
    x = hidden_states.reshape(-1, h)
    e, r = lora_A.shape[0], lora_A.shape[1]
    a_cat = lora_A.reshape(e * r, h)
    b_cat = lora_B.transpose(0, 2, 1).reshape(e * r, h)
    out, logits = _moe_block(x, router_w, a_cat, b_cat, w1, w2, w3)
    return out.reshape(b, s, h), logits


# speedup vs baseline: 1.0355x; 1.0095x over previous
import functools

import jax
import jax.numpy as jnp
from jax import lax
from jax.experimental import pallas as pl
from jax.experimental.pallas import tpu as pltpu
from jax.experimental.pallas import tpu_sc as plsc

_SCALING = 32.0 / 16.0
_E = 8
_R = 16
_NCORES = 1
_NW = 16 * _NCORES
_LANES = 16


def _router_body(x_ref, rw8_ref, logits_ref, logits_t_ref):
    xt = x_ref[...]
    logits_t_ref[...] = jax.lax.dot_general(
        rw8_ref[...], xt, (((1,), (1,)), ((), ())),
        preferred_element_type=jnp.float32)
    logits_ref[...] = jax.lax.dot_general(
        xt, rw8_ref[...], (((1,), (1,)), ((), ())),
        preferred_element_type=jnp.float32)


def _sc_routing_body(lg_hbm, w_hbm, lv, wv):
    chunk = lv.shape[1]
    wid = lax.axis_index("s") * _NCORES + lax.axis_index("c")
    base = wid * chunk
    pltpu.sync_copy(lg_hbm.at[:, pl.ds(base, chunk)], lv)
    for v in range(chunk // _LANES):
        sl = pl.ds(v * _LANES, _LANES)
        ls = [lv[e, sl] for e in range(_E)]
        m1 = ls[0]
        for e in range(1, _E):
            m1 = jnp.maximum(m1, ls[e])
        big = 1 << 30
        i1 = jnp.where(ls[0] == m1, 0, big)
        for e in range(1, _E):
            i1 = jnp.minimum(i1, jnp.where(ls[e] == m1, e, big))
        neg = jnp.float32(-3.0e38)
        l2 = [jnp.where(i1 == e, neg, ls[e]) for e in range(_E)]
        m2 = l2[0]
        for e in range(1, _E):
            m2 = jnp.maximum(m2, l2[e])
        i2 = jnp.where(l2[0] == m2, 0, big)
        for e in range(1, _E):
            i2 = jnp.minimum(i2, jnp.where(l2[e] == m2, e, big))
        s1 = 1.0 / (1.0 + jnp.exp(m2 - m1))
        s2 = 1.0 - s1
        for e in range(_E):
            w = jnp.where(i1 == e, s1, 0.0) + jnp.where(i2 == e, s2, 0.0)
            wv[e, sl] = w * _SCALING
    pltpu.sync_copy(wv, w_hbm.at[:, pl.ds(base, chunk)])


def _mlp_body(x_ref, acat_ref, w1_ref, w3_ref, w2_ref, w8_ref, p_ref,
              bcat_ref, out_ref, a_scr):
    f = pl.program_id(1)
    nf = pl.num_programs(1)
    xt = x_ref[...]

    @pl.when(f == 0)
    def _lora_a():
        a_scr[...] = jnp.dot(xt, acat_ref[...].T,
                             preferred_element_type=jnp.float32)

    dn = (((1,), (1,)), ((), ()))
    h1 = jax.lax.dot_general(xt, w1_ref[...], dn,
                             preferred_element_type=jnp.float32)
    h3 = jax.lax.dot_general(xt, w3_ref[...], dn,
                             preferred_element_type=jnp.float32)
    h = jax.nn.silu(h1) * h3
    contrib = jax.lax.dot_general(h, w2_ref[...], dn,
                                  preferred_element_type=jnp.float32)

    @pl.when(f == 0)
    def _init():
        out_ref[...] = contrib

    @pl.when(f != 0)
    def _acc():
        out_ref[...] += contrib

    @pl.when(f == nf - 1)
    def _combine():
        wl = jax.lax.dot_general(w8_ref[...], p_ref[...],
                                 (((0,), (0,)), ((), ())),
                                 preferred_element_type=jnp.float32)
        out_ref[...] += jax.lax.dot_general(
            a_scr[...] * wl, bcat_ref[...], (((1,), (0,)), ((), ())),
            preferred_element_type=jnp.float32)


@jax.jit
def _moe_block(x, rw8, a_cat, b_cat, w1, w2, w3):
    t, h = x.shape
    ffn = w1.shape[0]
    tm, fk = 1024, 1024
    nt, nf = t // tm, ffn // fk

    logits, logits_t = pl.pallas_call(
        _router_body,
        grid=(nt,),
        in_specs=[
            pl.BlockSpec((tm, h), lambda t_: (t_, 0)),
            pl.BlockSpec((_E, h), lambda t_: (0, 0)),
        ],
        out_specs=[
            pl.BlockSpec((tm, _E), lambda t_: (t_, 0)),
            pl.BlockSpec((_E, tm), lambda t_: (0, t_)),
        ],
        out_shape=[
            jax.ShapeDtypeStruct((t, _E), jnp.float32),
            jax.ShapeDtypeStruct((_E, t), jnp.float32),
        ],
    )(x, rw8)

    chunk = t // _NW
    mesh = plsc.VectorSubcoreMesh(core_axis_name="c", subcore_axis_name="s",
                                  num_cores=_NCORES)
    w8 = pl.kernel(
        _sc_routing_body,
        out_type=jax.ShapeDtypeStruct((_E, t), jnp.float32),
        mesh=mesh,
        scratch_types=[
            pltpu.VMEM((_E, chunk), jnp.float32),
            pltpu.VMEM((_E, chunk), jnp.float32),
        ],
    )(logits_t)

    p_sel = jnp.repeat(jnp.eye(_E, dtype=jnp.float32), _R, axis=1)
    out = pl.pallas_call(
        _mlp_body,
        grid=(nt, nf),
        in_specs=[
            pl.BlockSpec((tm, h), lambda t_, f_: (t_, 0)),
            pl.BlockSpec((128, h), lambda t_, f_: (0, 0)),
            pl.BlockSpec((fk, h), lambda t_, f_: (f_, 0)),
            pl.BlockSpec((fk, h), lambda t_, f_: (f_, 0)),
            pl.BlockSpec((h, fk), lambda t_, f_: (0, f_)),
            pl.BlockSpec((_E, tm), lambda t_, f_: (0, t_)),
            pl.BlockSpec((_E, 128), lambda t_, f_: (0, 0)),
            pl.BlockSpec((128, h), lambda t_, f_: (0, 0)),
        ],
        out_specs=pl.BlockSpec((tm, h), lambda t_, f_: (t_, 0)),
        out_shape=jax.ShapeDtypeStruct((t, h), jnp.float32),
        scratch_shapes=[pltpu.VMEM((tm, 128), jnp.float32)],
        compiler_params=pltpu.CompilerParams(
            dimension_semantics=("parallel", "arbitrary"),
        ),
    )(x, a_cat, w1, w3, w2, w8, p_sel, b_cat)
    return out, logits


def kernel(hidden_states, router_w, w1, w2, w3, lora_A, lora_B):
    b, s, h = hidden_states.shape
    x = hidden_states.reshape(-1, h)
    e, r = lora_A.shape[0], lora_A.shape[1]
    a_cat = lora_A.reshape(e * r, h)
    b_cat = lora_B.transpose(0, 2, 1).reshape(e * r, h)
    out, logits = _moe_block(x, router_w, a_cat, b_cat, w1, w2, w3)
    return out.reshape(b, s, h), logits


# speedup vs baseline: 1.0440x; 1.0082x over previous
import jax
import jax.numpy as jnp
from jax import lax
from jax.experimental import pallas as pl
from jax.experimental.pallas import tpu as pltpu
from jax.experimental.pallas import tpu_sc as plsc

_SCALING = 32.0 / 16.0
_E = 8
_R = 16
_NCORES = 1
_NW = 16 * _NCORES
_LANES = 16


def _router_body(x_ref, rw8_ref, logits_ref, logits_t_ref):
    xt = x_ref[...]
    logits_t_ref[...] = jax.lax.dot_general(
        rw8_ref[...], xt, (((1,), (1,)), ((), ())),
        preferred_element_type=jnp.float32)
    logits_ref[...] = jax.lax.dot_general(
        xt, rw8_ref[...], (((1,), (1,)), ((), ())),
        preferred_element_type=jnp.float32)


def _sc_routing_body(lg_hbm, w_hbm, lv, wv):
    chunk = lv.shape[1]
    wid = lax.axis_index("s") * _NCORES + lax.axis_index("c")
    base = wid * chunk
    pltpu.sync_copy(lg_hbm.at[:, pl.ds(base, chunk)], lv)
    for v in range(chunk // _LANES):
        sl = pl.ds(v * _LANES, _LANES)
        ls = [lv[e, sl] for e in range(_E)]
        m1 = ls[0]
        for e in range(1, _E):
            m1 = jnp.maximum(m1, ls[e])
        big = 1 << 30
        i1 = jnp.where(ls[0] == m1, 0, big)
        for e in range(1, _E):
            i1 = jnp.minimum(i1, jnp.where(ls[e] == m1, e, big))
        neg = jnp.float32(-3.0e38)
        l2 = [jnp.where(i1 == e, neg, ls[e]) for e in range(_E)]
        m2 = l2[0]
        for e in range(1, _E):
            m2 = jnp.maximum(m2, l2[e])
        i2 = jnp.where(l2[0] == m2, 0, big)
        for e in range(1, _E):
            i2 = jnp.minimum(i2, jnp.where(l2[e] == m2, e, big))
        s1 = 1.0 / (1.0 + jnp.exp(m2 - m1))
        s2 = 1.0 - s1
        for e in range(_E):
            w = jnp.where(i1 == e, s1, 0.0) + jnp.where(i2 == e, s2, 0.0)
            wv[e, sl] = w * _SCALING
    pltpu.sync_copy(wv, w_hbm.at[:, pl.ds(base, chunk)])


def _mlp_body(x_ref, acat_ref, w1_ref, w3_ref, w2_ref, w8_ref, p_ref,
              bcat_ref, out_ref, a_scr):
    f = pl.program_id(1)
    nf = pl.num_programs(1)
    xt = x_ref[...]

    @pl.when(f == 0)
    def _lora_a():
        a_scr[...] = jnp.dot(xt, acat_ref[...].T,
                             preferred_element_type=jnp.float32)

    dn = (((1,), (1,)), ((), ()))
    h1 = jax.lax.dot_general(xt, w1_ref[...], dn,
                             preferred_element_type=jnp.float32)
    h3 = jax.lax.dot_general(xt, w3_ref[...], dn,
                             preferred_element_type=jnp.float32)
    h = jax.nn.silu(h1) * h3
    contrib = jax.lax.dot_general(h, w2_ref[...], dn,
                                  preferred_element_type=jnp.float32)

    @pl.when(f == 0)
    def _init():
        out_ref[...] = contrib

    @pl.when(f != 0)
    def _acc():
        out_ref[...] += contrib

    @pl.when(f == nf - 1)
    def _combine():
        wl = jax.lax.dot_general(w8_ref[...], p_ref[...],
                                 (((0,), (0,)), ((), ())),
                                 preferred_element_type=jnp.float32)
        out_ref[...] += jax.lax.dot_general(
            a_scr[...] * wl, bcat_ref[...], (((1,), (0,)), ((), ())),
            preferred_element_type=jnp.float32)


@jax.jit
def _moe_block(x, rw8, a_cat, b_cat, w1, w2, w3):
    t, h = x.shape
    ffn = w1.shape[0]
    tm, fk = 1024, 1024
    nt, nf = t // tm, ffn // fk

    logits, logits_t = pl.pallas_call(
        _router_body,
        grid=(nt,),
        in_specs=[
            pl.BlockSpec((tm, h), lambda t_: (t_, 0)),
            pl.BlockSpec((_E, h), lambda t_: (0, 0)),
        ],
        out_specs=[
            pl.BlockSpec((tm, _E), lambda t_: (t_, 0)),
            pl.BlockSpec((_E, tm), lambda t_: (0, t_)),
        ],
        out_shape=[
            jax.ShapeDtypeStruct((t, _E), jnp.float32),
            jax.ShapeDtypeStruct((_E, t), jnp.float32),
        ],
    )(x, rw8)

    chunk = t // _NW
    mesh = plsc.VectorSubcoreMesh(core_axis_name="c", subcore_axis_name="s",
                                  num_cores=_NCORES)
    w8 = pl.kernel(
        _sc_routing_body,
        out_type=jax.ShapeDtypeStruct((_E, t), jnp.float32),
        mesh=mesh,
        scratch_types=[
            pltpu.VMEM((_E, chunk), jnp.float32),
            pltpu.VMEM((_E, chunk), jnp.float32),
        ],
    )(logits_t)

    p_sel = jnp.repeat(jnp.eye(_E, dtype=jnp.float32), _R, axis=1)
    out = pl.pallas_call(
        _mlp_body,
        grid=(nt, nf),
        in_specs=[
            pl.BlockSpec((tm, h), lambda t_, f_: (t_, 0)),
            pl.BlockSpec((128, h), lambda t_, f_: (0, 0)),
            pl.BlockSpec((fk, h), lambda t_, f_: (f_, 0)),
            pl.BlockSpec((fk, h), lambda t_, f_: (f_, 0)),
            pl.BlockSpec((h, fk), lambda t_, f_: (0, f_)),
            pl.BlockSpec((_E, tm), lambda t_, f_: (0, t_)),
            pl.BlockSpec((_E, 128), lambda t_, f_: (0, 0)),
            pl.BlockSpec((128, h), lambda t_, f_: (0, 0)),
        ],
        out_specs=pl.BlockSpec((tm, h), lambda t_, f_: (t_, 0)),
        out_shape=jax.ShapeDtypeStruct((t, h), jnp.float32),
        scratch_shapes=[pltpu.VMEM((tm, 128), jnp.float32)],
        compiler_params=pltpu.CompilerParams(
            dimension_semantics=("parallel", "arbitrary"),
        ),
    )(x, a_cat, w1, w3, w2, w8, p_sel, b_cat)
    return out, logits


def kernel(hidden_states, router_w, w1, w2, w3, lora_A, lora_B):
    b, s, h = hidden_states.shape
    x = hidden_states.reshape(-1, h)
    e, r = lora_A.shape[0], lora_A.shape[1]
    a_cat = lora_A.reshape(e * r, h)
    b_cat = lora_B.transpose(0, 2, 1).reshape(e * r, h)
    out, logits = _moe_block(x, router_w, a_cat, b_cat, w1, w2, w3)
    return out.reshape(b, s, h), logits
